# Initial kernel scaffold; baseline (speedup 1.0000x reference)
import jax
import jax.numpy as jnp
from jax.experimental import pallas as pl


def kernel(v0, adj_t, W1, b1, W2, b2):
    raise NotImplementedError("write your pallas kernel here")


# speedup vs baseline: 4.9346x; 4.9346x over previous
import functools

import jax
import jax.numpy as jnp
from jax import lax
from jax.experimental import pallas as pl
from jax.experimental.pallas import tpu as pltpu
from jax.experimental.pallas import tpu_sc as plsc

_N = 10000
_E = 320000
_D = 128

_NC = 2
_NS = 16
_NW = _NC * _NS

_CH = 80
_EPW = _E // _NW
_NCHUNK = _EPW // _CH

_ZCH = 16
_NZC = _N // _ZCH
_ZITER = (_NZC + _NS - 1) // _NS


def _sc_agg_body(h_hbm, src_hbm, dst_hbm, out_hbm,
                 src_v, dst_v, rows_v, zer_v, acc, sem):
    c = lax.axis_index("c")
    s = lax.axis_index("s")

    for i in range(_ZCH):
        for j in range(_D // 16):
            zer_v[i, pl.ds(j * 16, 16)] = jnp.zeros((16,), jnp.float32)

    def zero_body(j, carry):
        k = j * _NS + s
        @pl.when(k < _NZC)
        def _():
            pltpu.sync_copy(zer_v, acc.at[pl.ds(k * _ZCH, _ZCH)])
        return carry

    lax.fori_loop(0, _ZITER, zero_body, 0)
    plsc.subcore_barrier()

    wid = s * _NC + c
    base = wid * _EPW

    def edge_body(i, carry):
        off = base + i * _CH
        pltpu.sync_copy(src_hbm.at[pl.ds(off, _CH)], src_v)
        pltpu.sync_copy(dst_hbm.at[pl.ds(off, _CH)], dst_v)
        pltpu.async_copy(h_hbm.at[src_v], rows_v, sem).wait()
        pltpu.sync_copy(rows_v, acc.at[dst_v], add=True)
        return carry

    lax.fori_loop(0, _NCHUNK, edge_body, 0)
    plsc.subcore_barrier()

    def out_body(j, carry):
        k = j * _NS + s
        @pl.when(k < _NZC)
        def _():
            pltpu.sync_copy(acc.at[pl.ds(k * _ZCH, _ZCH)],
                            out_hbm.at[c, pl.ds(k * _ZCH, _ZCH)])
        return carry

    lax.fori_loop(0, _ZITER, out_body, 0)


_sc_agg = functools.partial(
    pl.kernel,
    out_type=jax.ShapeDtypeStruct((_NC, _N, _D), jnp.float32),
    mesh=plsc.VectorSubcoreMesh(core_axis_name="c", subcore_axis_name="s"),
    scratch_types=[
        pltpu.VMEM((_CH,), jnp.int32),
        pltpu.VMEM((_CH,), jnp.int32),
        pltpu.VMEM((_CH, _D), jnp.float32),
        pltpu.VMEM((_ZCH, _D), jnp.float32),
        pltpu.VMEM_SHARED((_N, _D), jnp.float32),
        pltpu.SemaphoreType.DMA,
    ],
)(_sc_agg_body)


_BLK = 2000
_GRID = _N // _BLK


def _mm_body(x_ref, w_ref, o_ref):
    o_ref[...] = jnp.dot(x_ref[...], w_ref[...],
                         preferred_element_type=jnp.float32)


def _layer2_body(p_ref, b_ref, w_ref, o_ref):
    x = p_ref[0] + p_ref[1] + b_ref[...]
    x = jnp.maximum(x, 0.0)
    o_ref[...] = jnp.dot(x, w_ref[...], preferred_element_type=jnp.float32)


def _combine_body(p_ref, b_ref, o_ref):
    o_ref[...] = p_ref[0] + p_ref[1] + b_ref[...]


def _mm(x, w):
    return pl.pallas_call(
        _mm_body,
        grid=(_GRID,),
        in_specs=[
            pl.BlockSpec((_BLK, _D), lambda i: (i, 0)),
            pl.BlockSpec((_D, _D), lambda i: (0, 0)),
        ],
        out_specs=pl.BlockSpec((_BLK, _D), lambda i: (i, 0)),
        out_shape=jax.ShapeDtypeStruct((_N, _D), jnp.float32),
    )(x, w)


def _layer2(p, b, w):
    return pl.pallas_call(
        _layer2_body,
        grid=(_GRID,),
        in_specs=[
            pl.BlockSpec((_NC, _BLK, _D), lambda i: (0, i, 0)),
            pl.BlockSpec((1, _D), lambda i: (0, 0)),
            pl.BlockSpec((_D, _D), lambda i: (0, 0)),
        ],
        out_specs=pl.BlockSpec((_BLK, _D), lambda i: (i, 0)),
        out_shape=jax.ShapeDtypeStruct((_N, _D), jnp.float32),
    )(p, b, w)


def _combine(p, b):
    return pl.pallas_call(
        _combine_body,
        grid=(_GRID,),
        in_specs=[
            pl.BlockSpec((_NC, _BLK, _D), lambda i: (0, i, 0)),
            pl.BlockSpec((1, _D), lambda i: (0, 0)),
        ],
        out_specs=pl.BlockSpec((_BLK, _D), lambda i: (i, 0)),
        out_shape=jax.ShapeDtypeStruct((_N, _D), jnp.float32),
    )(p, b)


def kernel(v0, adj_t, W1, b1, W2, b2):
    src = adj_t[0].astype(jnp.int32)
    dst = adj_t[1].astype(jnp.int32)
    b1r = b1.reshape(1, _D)
    b2r = b2.reshape(1, _D)

    h1 = _mm(v0, W1)
    p1 = _sc_agg(h1, src, dst)
    h2 = _layer2(p1, b1r, W2)
    p2 = _sc_agg(h2, src, dst)
    return _combine(p2, b2r)


# speedup vs baseline: 14.3217x; 2.9023x over previous
import functools

import jax
import jax.numpy as jnp
from jax import lax
from jax.experimental import pallas as pl
from jax.experimental.pallas import tpu as pltpu
from jax.experimental.pallas import tpu_sc as plsc

_N = 10000
_E = 320000
_D = 128

_NC = 2
_NS = 16
_NW = _NC * _NS

_CH = 80
_EPW = _E // _NW
_NCHUNK = _EPW // _CH
_NBUF = 3
_NDST = 6
_NSTEP = 6
_NOUT = (_NCHUNK + _NSTEP - 1) // _NSTEP

_OCH = 200
_NOC = _N // _OCH


def _sc_agg_body(h_hbm, src_hbm, dstr_hbm, out_hbm,
                 src_vv, rows0, rows1, rows2,
                 dst0, dst1, dst2, dst3, dst4, dst5,
                 acc, gsem0, gsem1, gsem2,
                 dsem0, dsem1, dsem2, dsem3, dsem4, dsem5):
    c = lax.axis_index("c")
    s = lax.axis_index("s")
    wid = s * _NC + c
    rows = (rows0, rows1, rows2)
    gsems = (gsem0, gsem1, gsem2)
    dsts = (dst0, dst1, dst2, dst3, dst4, dst5)
    dsems = (dsem0, dsem1, dsem2, dsem3, dsem4, dsem5)

    def zf_body(i, carry):
        for j in range(_D // 16):
            rows0[i, pl.ds(j * 16, 16)] = jnp.zeros((16,), jnp.float32)
        return carry

    lax.fori_loop(0, _CH, zf_body, 0)

    def zero_body(j, carry):
        k = j * _NS + s
        @pl.when(k < _NCHUNK)
        def _():
            pltpu.sync_copy(rows0, acc.at[pl.ds(k * _CH, _CH)])
        return carry

    lax.fori_loop(0, (_NCHUNK + _NS - 1) // _NS, zero_body, 0)
    plsc.subcore_barrier()

    base = wid * _EPW
    pltpu.sync_copy(src_hbm.at[pl.ds(base, _EPW)], src_vv)

    for d in range(_NDST):
        pltpu.async_copy(dstr_hbm.at[wid, d], dsts[d], dsems[d])
    for b in range(_NBUF):
        pltpu.async_copy(h_hbm.at[src_vv.at[pl.ds(b * _CH, _CH)]],
                         rows[b], gsems[b])

    def edge_body(g, carry):
        for t in range(_NSTEP):
            i = g * _NSTEP + t
            b = t % _NBUF
            d = t % _NDST
            @pl.when(i < _NCHUNK)
            def _():
                pltpu.make_async_copy(
                    h_hbm.at[src_vv.at[pl.ds(i * _CH, _CH)]],
                    rows[b], gsems[b]).wait()
                pltpu.make_async_copy(dstr_hbm.at[wid, d], dsts[d],
                                      dsems[d]).wait()
                pltpu.sync_copy(rows[b], acc.at[dsts[d]], add=True)
                @pl.when(i + _NBUF < _NCHUNK)
                def _():
                    pltpu.async_copy(
                        h_hbm.at[src_vv.at[pl.ds((i + _NBUF) * _CH, _CH)]],
                        rows[b], gsems[b])
                @pl.when(i + _NDST < _NCHUNK)
                def _():
                    pltpu.async_copy(dstr_hbm.at[wid, i + _NDST], dsts[d],
                                     dsems[d])
        return carry

    lax.fori_loop(0, _NOUT, edge_body, 0)
    plsc.subcore_barrier()

    def out_body(j, carry):
        k = j * _NS + s
        @pl.when(k < _NOC)
        def _():
            pltpu.sync_copy(acc.at[pl.ds(k * _OCH, _OCH)],
                            out_hbm.at[c, pl.ds(k * _OCH, _OCH)])
        return carry

    lax.fori_loop(0, (_NOC + _NS - 1) // _NS, out_body, 0)


_sc_agg = functools.partial(
    pl.kernel,
    out_type=jax.ShapeDtypeStruct((_NC, _N, _D), jnp.float32),
    mesh=plsc.VectorSubcoreMesh(core_axis_name="c", subcore_axis_name="s"),
    scratch_types=[
        pltpu.VMEM((_EPW,), jnp.int32),
        pltpu.VMEM((_CH, _D), jnp.float32),
        pltpu.VMEM((_CH, _D), jnp.float32),
        pltpu.VMEM((_CH, _D), jnp.float32),
        pltpu.VMEM((_CH,), jnp.int32),
        pltpu.VMEM((_CH,), jnp.int32),
        pltpu.VMEM((_CH,), jnp.int32),
        pltpu.VMEM((_CH,), jnp.int32),
        pltpu.VMEM((_CH,), jnp.int32),
        pltpu.VMEM((_CH,), jnp.int32),
        pltpu.VMEM_SHARED((_N, _D), jnp.float32),
        pltpu.SemaphoreType.DMA,
        pltpu.SemaphoreType.DMA,
        pltpu.SemaphoreType.DMA,
        pltpu.SemaphoreType.DMA,
        pltpu.SemaphoreType.DMA,
        pltpu.SemaphoreType.DMA,
        pltpu.SemaphoreType.DMA,
        pltpu.SemaphoreType.DMA,
        pltpu.SemaphoreType.DMA,
    ],
)(_sc_agg_body)


_BLK = 2000
_GRID = _N // _BLK


def _mm_body(x_ref, w_ref, o_ref):
    o_ref[...] = jnp.dot(x_ref[...], w_ref[...],
                         preferred_element_type=jnp.float32)


def _layer2_body(p_ref, b_ref, w_ref, o_ref):
    x = p_ref[0] + p_ref[1] + b_ref[...]
    x = jnp.maximum(x, 0.0)
    o_ref[...] = jnp.dot(x, w_ref[...], preferred_element_type=jnp.float32)


def _combine_body(p_ref, b_ref, o_ref):
    o_ref[...] = p_ref[0] + p_ref[1] + b_ref[...]


def _mm(x, w):
    return pl.pallas_call(
        _mm_body,
        grid=(_GRID,),
        in_specs=[
            pl.BlockSpec((_BLK, _D), lambda i: (i, 0)),
            pl.BlockSpec((_D, _D), lambda i: (0, 0)),
        ],
        out_specs=pl.BlockSpec((_BLK, _D), lambda i: (i, 0)),
        out_shape=jax.ShapeDtypeStruct((_N, _D), jnp.float32),
    )(x, w)


def _layer2(p, b, w):
    return pl.pallas_call(
        _layer2_body,
        grid=(_GRID,),
        in_specs=[
            pl.BlockSpec((_NC, _BLK, _D), lambda i: (0, i, 0)),
            pl.BlockSpec((1, _D), lambda i: (0, 0)),
            pl.BlockSpec((_D, _D), lambda i: (0, 0)),
        ],
        out_specs=pl.BlockSpec((_BLK, _D), lambda i: (i, 0)),
        out_shape=jax.ShapeDtypeStruct((_N, _D), jnp.float32),
    )(p, b, w)


def _combine(p, b):
    return pl.pallas_call(
        _combine_body,
        grid=(_GRID,),
        in_specs=[
            pl.BlockSpec((_NC, _BLK, _D), lambda i: (0, i, 0)),
            pl.BlockSpec((1, _D), lambda i: (0, 0)),
        ],
        out_specs=pl.BlockSpec((_BLK, _D), lambda i: (i, 0)),
        out_shape=jax.ShapeDtypeStruct((_N, _D), jnp.float32),
    )(p, b)


def kernel(v0, adj_t, W1, b1, W2, b2):
    src = adj_t[0].astype(jnp.int32)
    dst = adj_t[1].astype(jnp.int32).reshape(_NW, _NCHUNK, _CH)
    b1r = b1.reshape(1, _D)
    b2r = b2.reshape(1, _D)

    h1 = _mm(v0, W1)
    p1 = _sc_agg(h1, src, dst)
    h2 = _layer2(p1, b1r, W2)
    p2 = _sc_agg(h2, src, dst)
    return _combine(p2, b2r)


# speedup vs baseline: 15.0949x; 1.0540x over previous
import functools

import jax
import jax.numpy as jnp
from jax import lax
from jax.experimental import pallas as pl
from jax.experimental.pallas import tpu as pltpu
from jax.experimental.pallas import tpu_sc as plsc

_N = 10000
_E = 320000
_D = 128

_NC = 2
_NS = 16
_NW = _NC * _NS

_CH = 80
_EPW = _E // _NW
_NCHUNK = _EPW // _CH
_NBUF = 3
_NDST = 6
_NSTEP = 6
_NOUT = (_NCHUNK + _NSTEP - 1) // _NSTEP

_ZCH = 40
_NZC = _N // _ZCH
_ZIT = (_NZC + _NS - 1) // _NS

_OCH = 200
_NOC = _N // _OCH
_OIT = (_NOC + _NS - 1) // _NS


def _sc_agg_body(h_hbm, src_hbm, dstr_hbm, out_hbm,
                 src_vv, rows0, rows1, rows2, zbuf,
                 dst0, dst1, dst2, dst3, dst4, dst5,
                 acc, gsem0, gsem1, gsem2,
                 dsem0, dsem1, dsem2, dsem3, dsem4, dsem5,
                 zsem, osem):
    c = lax.axis_index("c")
    s = lax.axis_index("s")
    wid = s * _NC + c
    rows = (rows0, rows1, rows2)
    gsems = (gsem0, gsem1, gsem2)
    dsts = (dst0, dst1, dst2, dst3, dst4, dst5)
    dsems = (dsem0, dsem1, dsem2, dsem3, dsem4, dsem5)

    def zf_body(i, carry):
        for j in range(_D // 16):
            zbuf[i, pl.ds(j * 16, 16)] = jnp.zeros((16,), jnp.float32)
        return carry

    lax.fori_loop(0, _ZCH, zf_body, 0)

    def zfire_body(j, carry):
        k = j * _NS + s
        @pl.when(k < _NZC)
        def _():
            pltpu.async_copy(zbuf, acc.at[pl.ds(k * _ZCH, _ZCH)], zsem)
        return carry

    lax.fori_loop(0, _ZIT, zfire_body, 0)

    base = wid * _EPW
    pltpu.sync_copy(src_hbm.at[pl.ds(base, _EPW)], src_vv)
    for d in range(_NDST):
        pltpu.async_copy(dstr_hbm.at[wid, d], dsts[d], dsems[d])
    for b in range(_NBUF):
        pltpu.async_copy(h_hbm.at[src_vv.at[pl.ds(b * _CH, _CH)]],
                         rows[b], gsems[b])

    def zdrain_body(j, carry):
        k = j * _NS + s
        @pl.when(k < _NZC)
        def _():
            pltpu.make_async_copy(zbuf, acc.at[pl.ds(k * _ZCH, _ZCH)],
                                  zsem).wait()
        return carry

    lax.fori_loop(0, _ZIT, zdrain_body, 0)
    plsc.subcore_barrier()

    def edge_body(g, carry):
        for t in range(_NSTEP):
            i = g * _NSTEP + t
            b = t % _NBUF
            d = t % _NDST
            @pl.when(i < _NCHUNK)
            def _():
                pltpu.make_async_copy(
                    h_hbm.at[src_vv.at[pl.ds(i * _CH, _CH)]],
                    rows[b], gsems[b]).wait()
                pltpu.make_async_copy(dstr_hbm.at[wid, d], dsts[d],
                                      dsems[d]).wait()
                pltpu.sync_copy(rows[b], acc.at[dsts[d]], add=True)
                @pl.when(i + _NBUF < _NCHUNK)
                def _():
                    pltpu.async_copy(
                        h_hbm.at[src_vv.at[pl.ds((i + _NBUF) * _CH, _CH)]],
                        rows[b], gsems[b])
                @pl.when(i + _NDST < _NCHUNK)
                def _():
                    pltpu.async_copy(dstr_hbm.at[wid, i + _NDST], dsts[d],
                                     dsems[d])
        return carry

    lax.fori_loop(0, _NOUT, edge_body, 0)
    plsc.subcore_barrier()

    def ofire_body(j, carry):
        k = j * _NS + s
        @pl.when(k < _NOC)
        def _():
            pltpu.async_copy(acc.at[pl.ds(k * _OCH, _OCH)],
                             out_hbm.at[c, pl.ds(k * _OCH, _OCH)], osem)
        return carry

    lax.fori_loop(0, _OIT, ofire_body, 0)

    def odrain_body(j, carry):
        k = j * _NS + s
        @pl.when(k < _NOC)
        def _():
            pltpu.make_async_copy(acc.at[pl.ds(k * _OCH, _OCH)],
                                  out_hbm.at[c, pl.ds(k * _OCH, _OCH)],
                                  osem).wait()
        return carry

    lax.fori_loop(0, _OIT, odrain_body, 0)


_sc_agg = functools.partial(
    pl.kernel,
    out_type=jax.ShapeDtypeStruct((_NC, _N, _D), jnp.float32),
    mesh=plsc.VectorSubcoreMesh(core_axis_name="c", subcore_axis_name="s"),
    scratch_types=[
        pltpu.VMEM((_EPW,), jnp.int32),
        pltpu.VMEM((_CH, _D), jnp.float32),
        pltpu.VMEM((_CH, _D), jnp.float32),
        pltpu.VMEM((_CH, _D), jnp.float32),
        pltpu.VMEM((_ZCH, _D), jnp.float32),
        pltpu.VMEM((_CH,), jnp.int32),
        pltpu.VMEM((_CH,), jnp.int32),
        pltpu.VMEM((_CH,), jnp.int32),
        pltpu.VMEM((_CH,), jnp.int32),
        pltpu.VMEM((_CH,), jnp.int32),
        pltpu.VMEM((_CH,), jnp.int32),
        pltpu.VMEM_SHARED((_N, _D), jnp.float32),
        pltpu.SemaphoreType.DMA,
        pltpu.SemaphoreType.DMA,
        pltpu.SemaphoreType.DMA,
        pltpu.SemaphoreType.DMA,
        pltpu.SemaphoreType.DMA,
        pltpu.SemaphoreType.DMA,
        pltpu.SemaphoreType.DMA,
        pltpu.SemaphoreType.DMA,
        pltpu.SemaphoreType.DMA,
        pltpu.SemaphoreType.DMA,
        pltpu.SemaphoreType.DMA,
    ],
)(_sc_agg_body)


_BLK = 2000
_GRID = _N // _BLK


def _layer_body(relu, p_ref, b_ref, w_ref, o_ref):
    x = jnp.dot(p_ref[0] + p_ref[1], w_ref[...],
                preferred_element_type=jnp.float32) + b_ref[...]
    if relu:
        x = jnp.maximum(x, 0.0)
    o_ref[...] = x


def _layer(p, b, w, relu):
    return pl.pallas_call(
        functools.partial(_layer_body, relu),
        grid=(_GRID,),
        in_specs=[
            pl.BlockSpec((_NC, _BLK, _D), lambda i: (0, i, 0)),
            pl.BlockSpec((1, _D), lambda i: (0, 0)),
            pl.BlockSpec((_D, _D), lambda i: (0, 0)),
        ],
        out_specs=pl.BlockSpec((_BLK, _D), lambda i: (i, 0)),
        out_shape=jax.ShapeDtypeStruct((_N, _D), jnp.float32),
    )(p, b, w)


def kernel(v0, adj_t, W1, b1, W2, b2):
    src = adj_t[0].astype(jnp.int32)
    dst = adj_t[1].astype(jnp.int32).reshape(_NW, _NCHUNK, _CH)
    b1r = b1.reshape(1, _D)
    b2r = b2.reshape(1, _D)

    p1 = _sc_agg(v0, src, dst)
    x = _layer(p1, b1r, W1, relu=True)
    p2 = _sc_agg(x, src, dst)
    return _layer(p2, b2r, W2, relu=False)
